# Initial kernel scaffold; baseline (speedup 1.0000x reference)
import jax
import jax.numpy as jnp
from jax.experimental import pallas as pl


def kernel(features, edge_index, W1, b1, W2, b2, W3, b3):
    raise NotImplementedError("write your pallas kernel here")


# speedup vs baseline: 5.0468x; 5.0468x over previous
import jax
import jax.numpy as jnp
from jax import lax
from jax.experimental import pallas as pl
from jax.experimental.pallas import tpu as pltpu
from jax.experimental.pallas import tpu_sc as plsc

NN = 10000
EE = 160000
SB = 624
SL = 640
CB = 128

f32 = jnp.float32
i32 = jnp.int32


def _mesh():
    return plsc.VectorSubcoreMesh(core_axis_name="c", subcore_axis_name="s")


def _stripe(s, fn):
    @pl.when(s < 15)
    def _():
        fn(s * SB, SB)

    @pl.when(s == 15)
    def _():
        fn(15 * SB, SL)


def _deg_body(src_hbm, dst_hbm, out_hbm, hist_s, hist_d, idx_v, idx_v2):
    c = lax.axis_index("c")
    s = lax.axis_index("s")
    w = c * 16 + s
    zz = jnp.zeros((16,), f32)

    def zloop(i, _):
        hist_s[pl.ds(i * 16, 16)] = zz
        hist_d[pl.ds(i * 16, 16)] = zz
        return 0

    lax.fori_loop(0, NN // 16, zloop, 0)
    ones_r = jnp.ones((16,), f32)

    def run(ebase, nchunks):
        def chunk(k, _):
            b = ebase + k * CB
            pltpu.sync_copy(src_hbm.at[pl.ds(b, CB)], idx_v)
            pltpu.sync_copy(dst_hbm.at[pl.ds(b, CB)], idx_v2)
            for j in range(CB // 16):
                sl = pl.ds(j * 16, 16)
                plsc.addupdate_scatter(hist_s, [idx_v[sl]], ones_r)
                plsc.addupdate_scatter(hist_d, [idx_v2[sl]], ones_r)
            return 0

        lax.fori_loop(0, nchunks, chunk, 0)

    @pl.when(w < 31)
    def _():
        run(w * 4992, 39)

    @pl.when(w == 31)
    def _():
        run(31 * 4992, 41)

    pltpu.sync_copy(hist_s, out_hbm.at[pl.ds(w * NN, NN)])
    pltpu.sync_copy(hist_d, out_hbm.at[pl.ds((32 + w) * NN, NN)])


def _deg_call(src, dst):
    return pl.kernel(
        _deg_body,
        out_type=jax.ShapeDtypeStruct((64 * NN,), f32),
        mesh=_mesh(),
        compiler_params=pltpu.CompilerParams(needs_layout_passes=False),
        scratch_types=[
            pltpu.VMEM((NN,), f32),
            pltpu.VMEM((NN,), f32),
            pltpu.VMEM((CB,), i32),
            pltpu.VMEM((CB,), i32),
        ],
    )(src, dst)


def _agg_body(h_hbm, src_hbm, dst_hbm, out_hbm,
              acc, idx_v, dst_v, rows_v, sem):
    c = lax.axis_index("c")
    s = lax.axis_index("s")
    off = c * NN

    def initf(base, n):
        def ic(i, _):
            pltpu.sync_copy(h_hbm.at[pl.ds(off + base + i * CB, CB)], rows_v)
            pltpu.sync_copy(rows_v, acc.at[pl.ds(base + i * CB, CB)])
            return 0

        lax.fori_loop(0, n // CB, ic, 0)
        r = n % CB
        if r:
            o = (n // CB) * CB
            pltpu.sync_copy(h_hbm.at[pl.ds(off + base + o, r)],
                            rows_v.at[pl.ds(0, r)])
            pltpu.sync_copy(rows_v.at[pl.ds(0, r)], acc.at[pl.ds(base + o, r)])

    _stripe(s, initf)
    plsc.subcore_barrier()

    def run(ebase, nchunks):
        def chunk(k, _):
            b = ebase + k * CB
            pltpu.sync_copy(src_hbm.at[pl.ds(b, CB)], idx_v)
            pltpu.sync_copy(dst_hbm.at[pl.ds(b, CB)], dst_v)
            for j in range(CB // 16):
                sl = pl.ds(j * 16, 16)
                idx_v[sl] = idx_v[sl] + off
            pltpu.async_copy(h_hbm.at[idx_v], rows_v, sem).wait()
            pltpu.sync_copy(rows_v, acc.at[dst_v], add=True)
            return 0

        lax.fori_loop(0, nchunks, chunk, 0)

    @pl.when(s < 15)
    def _():
        run(s * 9984, 78)

    @pl.when(s == 15)
    def _():
        run(15 * 9984, 80)

    plsc.subcore_barrier()

    def drainf(base, n):
        def dc(i, _):
            pltpu.sync_copy(acc.at[pl.ds(base + i * CB, CB)], rows_v)
            pltpu.sync_copy(rows_v,
                            out_hbm.at[pl.ds(off + base + i * CB, CB)])
            return 0

        lax.fori_loop(0, n // CB, dc, 0)
        r = n % CB
        if r:
            o = (n // CB) * CB
            pltpu.sync_copy(acc.at[pl.ds(base + o, r)],
                            rows_v.at[pl.ds(0, r)])
            pltpu.sync_copy(rows_v.at[pl.ds(0, r)],
                            out_hbm.at[pl.ds(off + base + o, r)])

    _stripe(s, drainf)


def _agg_call(h_flat, src, dst):
    return pl.kernel(
        _agg_body,
        out_type=jax.ShapeDtypeStruct((2 * NN, 128), f32),
        mesh=_mesh(),
        scratch_types=[
            pltpu.VMEM_SHARED((NN, 128), f32),
            pltpu.VMEM((CB,), i32),
            pltpu.VMEM((CB,), i32),
            pltpu.VMEM((CB, 128), f32),
            pltpu.SemaphoreType.DMA,
        ],
    )(h_flat, src, dst)


def _src_norm(d):
    return lax.rsqrt(jnp.sum(d[:, 0:32], axis=1) + 1.0)[:, None]


def _dst_norm(d):
    return lax.rsqrt(jnp.sum(d[:, 32:64], axis=1) + 1.0)[:, None]


def _prep_body(feat_ref, deg_ref, out_ref):
    out_ref[...] = feat_ref[...] * _src_norm(deg_ref[...])


def _prep_call(features, degs):
    return pl.pallas_call(
        _prep_body,
        grid=(2, 10),
        in_specs=[
            pl.BlockSpec((1000, 128), lambda c, r: (r, c)),
            pl.BlockSpec((1000, 64), lambda c, r: (r, 0)),
        ],
        out_specs=pl.BlockSpec((1000, 128), lambda c, r: (c * 10 + r, 0)),
        out_shape=jax.ShapeDtypeStruct((2 * NN, 128), f32),
    )(features, degs)


def _layer_body(agg_ref, deg_ref, w_ref, b_ref, out_ref, acc_ref):
    k = pl.program_id(2)
    d = deg_ref[...]
    x = agg_ref[...] * _dst_norm(d)
    p = jnp.dot(x, w_ref[...], preferred_element_type=f32)

    @pl.when(k == 0)
    def _():
        acc_ref[...] = p

    @pl.when(k == 1)
    def _():
        a = acc_ref[...] + p + b_ref[0, 0]
        out_ref[...] = jnp.maximum(a, 0.0) * _src_norm(d)


def _layer_call(agg_flat, degs, W, b):
    return pl.pallas_call(
        _layer_body,
        grid=(2, 10, 2),
        in_specs=[
            pl.BlockSpec((1000, 128), lambda c, r, k: (k * 10 + r, 0)),
            pl.BlockSpec((1000, 64), lambda c, r, k: (r, 0)),
            pl.BlockSpec((128, 128), lambda c, r, k: (k, c)),
            pl.BlockSpec((1, 1, 128), lambda c, r, k: (c, 0, 0)),
        ],
        out_specs=pl.BlockSpec((1000, 128), lambda c, r, k: (c * 10 + r, 0)),
        out_shape=jax.ShapeDtypeStruct((2 * NN, 128), f32),
        scratch_shapes=[pltpu.VMEM((1000, 128), f32)],
    )(agg_flat, degs, W, b.reshape(2, 1, 128))


def _pre3_body(h_ref, w_ref, out_ref, acc_ref):
    k = pl.program_id(2)
    p = jnp.dot(h_ref[...], w_ref[...], preferred_element_type=f32)

    @pl.when(k == 0)
    def _():
        acc_ref[...] = p

    c = pl.program_id(0)

    @pl.when((k == 1) & (c == 0))
    def _():
        out_ref[...] = jnp.concatenate(
            [acc_ref[...] + p, jnp.zeros((1000, 64), f32)], axis=1)

    @pl.when((k == 1) & (c == 1))
    def _():
        out_ref[...] = jnp.zeros((1000, 128), f32)


def _pre3_call(h2s, W3):
    return pl.pallas_call(
        _pre3_body,
        grid=(2, 10, 2),
        in_specs=[
            pl.BlockSpec((1000, 128), lambda c, r, k: (k * 10 + r, 0)),
            pl.BlockSpec((128, 64), lambda c, r, k: (k, 0)),
        ],
        out_specs=pl.BlockSpec((1000, 128), lambda c, r, k: (c * 10 + r, 0)),
        out_shape=jax.ShapeDtypeStruct((2 * NN, 128), f32),
        scratch_shapes=[pltpu.VMEM((1000, 64), f32)],
    )(h2s, W3)


def _final_body(pa_ref, deg_ref, b_ref, out_ref):
    d = deg_ref[...]
    x = (pa_ref[...] * _dst_norm(d))[:, :64] + b_ref[0]
    out_ref[...] = 1.0 / (1.0 + jnp.exp(-x)) + 1e-8


def _final_call(p, degs, b3):
    return pl.pallas_call(
        _final_body,
        grid=(10,),
        in_specs=[
            pl.BlockSpec((1000, 128), lambda r: (r, 0)),
            pl.BlockSpec((1000, 64), lambda r: (r, 0)),
            pl.BlockSpec((1, 64), lambda r: (0, 0)),
        ],
        out_specs=pl.BlockSpec((1000, 64), lambda r: (r, 0)),
        out_shape=jax.ShapeDtypeStruct((NN, 64), f32),
    )(p, degs, b3.reshape(1, 64))


def kernel(features, edge_index, W1, b1, W2, b2, W3, b3):
    src = edge_index[0]
    dst = edge_index[1]

    degs = _deg_call(src, dst).reshape(64, NN).transpose(1, 0)
    h_s = _prep_call(features, degs)
    agg1 = _agg_call(h_s, src, dst)
    h1s = _layer_call(agg1, degs, W1, b1)
    agg2 = _agg_call(h1s, src, dst)
    h2s = _layer_call(agg2, degs, W2, b2)
    t = _pre3_call(h2s, W3)
    p = _agg_call(t, src, dst)
    return _final_call(p, degs, b3)


# speedup vs baseline: 7.7551x; 1.5366x over previous
import jax
import jax.numpy as jnp
from jax import lax
from jax.experimental import pallas as pl
from jax.experimental.pallas import tpu as pltpu
from jax.experimental.pallas import tpu_sc as plsc

NN = 10000
EE = 160000
SB = 624
SL = 640
CB = 128

f32 = jnp.float32
i32 = jnp.int32


def _mesh():
    return plsc.VectorSubcoreMesh(core_axis_name="c", subcore_axis_name="s")


def _stripe(s, fn):
    @pl.when(s < 15)
    def _():
        fn(s * SB, SB)

    @pl.when(s == 15)
    def _():
        fn(15 * SB, SL)


def _deg_body(src_hbm, dst_hbm, out_hbm, hist_s, hist_d, idx_v, idx_v2):
    c = lax.axis_index("c")
    s = lax.axis_index("s")
    w = c * 16 + s
    zz = jnp.zeros((16,), f32)

    def zloop(i, _):
        hist_s[pl.ds(i * 16, 16)] = zz
        hist_d[pl.ds(i * 16, 16)] = zz
        return 0

    lax.fori_loop(0, NN // 16, zloop, 0)
    ones_r = jnp.ones((16,), f32)

    def block(ebase, nedges):
        pltpu.sync_copy(src_hbm.at[pl.ds(ebase, nedges)],
                        idx_v.at[pl.ds(0, nedges)])
        pltpu.sync_copy(dst_hbm.at[pl.ds(ebase, nedges)],
                        idx_v2.at[pl.ds(0, nedges)])

        def group(i, _):
            sl = pl.ds(i * 16, 16)
            plsc.addupdate_scatter(hist_s, [idx_v[sl]], ones_r)
            plsc.addupdate_scatter(hist_d, [idx_v2[sl]], ones_r)
            return 0

        lax.fori_loop(0, nedges // 16, group, 0)

    def chunks(k, _):
        block(w * 4992 + k * 1248, 1248)
        return 0

    lax.fori_loop(0, 4, chunks, 0)

    @pl.when(w == 31)
    def _():
        block(31 * 4992 + 4992, 256)

    pltpu.sync_copy(hist_s, out_hbm.at[pl.ds(w * NN, NN)])
    pltpu.sync_copy(hist_d, out_hbm.at[pl.ds((32 + w) * NN, NN)])


def _deg_call(src, dst):
    return pl.kernel(
        _deg_body,
        out_type=jax.ShapeDtypeStruct((64 * NN,), f32),
        mesh=_mesh(),
        compiler_params=pltpu.CompilerParams(needs_layout_passes=False),
        scratch_types=[
            pltpu.VMEM((NN,), f32),
            pltpu.VMEM((NN,), f32),
            pltpu.VMEM((1248,), i32),
            pltpu.VMEM((1248,), i32),
        ],
    )(src, dst)


CBA = 80
KSC = 5


def _agg_body(h_hbm, src0_hbm, src1_hbm, dst_hbm, out_hbm,
              acc, idxb, dstb, rowsb, isem, dsem, g0, g1):
    c = lax.axis_index("c")
    s = lax.axis_index("s")
    off = c * NN
    gsems = (g0, g1)

    def initf(base, n):
        def ic(i, _):
            pltpu.sync_copy(h_hbm.at[pl.ds(off + base + i * CB, CB)],
                            rowsb.at[pl.ds(0, CB)])
            pltpu.sync_copy(rowsb.at[pl.ds(0, CB)],
                            acc.at[pl.ds(base + i * CB, CB)])
            return 0

        lax.fori_loop(0, n // CB, ic, 0)
        r = n % CB
        if r:
            o = (n // CB) * CB
            pltpu.sync_copy(h_hbm.at[pl.ds(off + base + o, r)],
                            rowsb.at[pl.ds(0, r)])
            pltpu.sync_copy(rowsb.at[pl.ds(0, r)], acc.at[pl.ds(base + o, r)])

    _stripe(s, initf)
    plsc.subcore_barrier()

    def superchunk(src_hbm, base):
        ic = pltpu.async_copy(src_hbm.at[pl.ds(base, KSC * CBA)],
                              idxb, isem)
        dcs = [pltpu.async_copy(dst_hbm.at[pl.ds(base + j * CBA, CBA)],
                                dstb.at[j], dsem) for j in range(KSC)]
        ic.wait()
        gws = [None] * KSC
        for j in range(2):
            gws[j] = pltpu.async_copy(
                h_hbm.at[idxb.at[pl.ds(j * CBA, CBA)]],
                rowsb.at[pl.ds((j % 2) * CBA, CBA)], gsems[j % 2])
        for d in dcs:
            d.wait()
        for j in range(KSC):
            gws[j].wait()
            pltpu.sync_copy(rowsb.at[pl.ds((j % 2) * CBA, CBA)],
                            acc.at[dstb.at[j]], add=True)
            if j + 2 < KSC:
                gws[j + 2] = pltpu.async_copy(
                    h_hbm.at[idxb.at[pl.ds((j + 2) * CBA, CBA)]],
                    rowsb.at[pl.ds((j % 2) * CBA, CBA)], gsems[j % 2])

    def run(src_hbm):
        ebase = s * 10000

        def sc_loop(g, _):
            superchunk(src_hbm, ebase + g * (KSC * CBA))
            return 0

        lax.fori_loop(0, 25, sc_loop, 0)

    @pl.when(c == 0)
    def _():
        run(src0_hbm)

    @pl.when(c == 1)
    def _():
        run(src1_hbm)

    plsc.subcore_barrier()

    def drainf(base, n):
        def dc(i, _):
            pltpu.sync_copy(acc.at[pl.ds(base + i * CB, CB)],
                            rowsb.at[pl.ds(0, CB)])
            pltpu.sync_copy(rowsb.at[pl.ds(0, CB)],
                            out_hbm.at[pl.ds(off + base + i * CB, CB)])
            return 0

        lax.fori_loop(0, n // CB, dc, 0)
        r = n % CB
        if r:
            o = (n // CB) * CB
            pltpu.sync_copy(acc.at[pl.ds(base + o, r)],
                            rowsb.at[pl.ds(0, r)])
            pltpu.sync_copy(rowsb.at[pl.ds(0, r)],
                            out_hbm.at[pl.ds(off + base + o, r)])

    _stripe(s, drainf)


def _agg_call(h_flat, src, src_plus, dst):
    return pl.kernel(
        _agg_body,
        out_type=jax.ShapeDtypeStruct((2 * NN, 128), f32),
        mesh=_mesh(),
        scratch_types=[
            pltpu.VMEM_SHARED((NN, 128), f32),
            pltpu.VMEM((KSC * CBA,), i32),
            pltpu.VMEM((KSC, CBA), i32),
            pltpu.VMEM((2 * CBA, 128), f32),
            pltpu.SemaphoreType.DMA,
            pltpu.SemaphoreType.DMA,
            pltpu.SemaphoreType.DMA,
            pltpu.SemaphoreType.DMA,
        ],
    )(h_flat, src, src_plus, dst)


def _src_norm(d):
    return lax.rsqrt(jnp.sum(d[:, 0:32], axis=1) + 1.0)[:, None]


def _dst_norm(d):
    return lax.rsqrt(jnp.sum(d[:, 32:64], axis=1) + 1.0)[:, None]


def _prep_body(feat_ref, deg_ref, out_ref):
    out_ref[...] = feat_ref[...] * _src_norm(deg_ref[...])


def _prep_call(features, degs):
    return pl.pallas_call(
        _prep_body,
        grid=(2, 10),
        in_specs=[
            pl.BlockSpec((1000, 128), lambda c, r: (r, c)),
            pl.BlockSpec((1000, 64), lambda c, r: (r, 0)),
        ],
        out_specs=pl.BlockSpec((1000, 128), lambda c, r: (c * 10 + r, 0)),
        out_shape=jax.ShapeDtypeStruct((2 * NN, 128), f32),
    )(features, degs)


def _layer_body(agg_ref, deg_ref, w_ref, b_ref, out_ref, acc_ref):
    k = pl.program_id(2)
    d = deg_ref[...]
    x = agg_ref[...] * _dst_norm(d)
    p = jnp.dot(x, w_ref[...], preferred_element_type=f32)

    @pl.when(k == 0)
    def _():
        acc_ref[...] = p

    @pl.when(k == 1)
    def _():
        a = acc_ref[...] + p + b_ref[0, 0]
        out_ref[...] = jnp.maximum(a, 0.0) * _src_norm(d)


def _layer_call(agg_flat, degs, W, b):
    return pl.pallas_call(
        _layer_body,
        grid=(2, 10, 2),
        in_specs=[
            pl.BlockSpec((1000, 128), lambda c, r, k: (k * 10 + r, 0)),
            pl.BlockSpec((1000, 64), lambda c, r, k: (r, 0)),
            pl.BlockSpec((128, 128), lambda c, r, k: (k, c)),
            pl.BlockSpec((1, 1, 128), lambda c, r, k: (c, 0, 0)),
        ],
        out_specs=pl.BlockSpec((1000, 128), lambda c, r, k: (c * 10 + r, 0)),
        out_shape=jax.ShapeDtypeStruct((2 * NN, 128), f32),
        scratch_shapes=[pltpu.VMEM((1000, 128), f32)],
    )(agg_flat, degs, W, b.reshape(2, 1, 128))


def _l2t_body(aggA_ref, aggB_ref, deg_ref, w2_ref, b2_ref, w3_ref, out_ref):
    c = pl.program_id(0)
    d = deg_ref[...]
    dn = _dst_norm(d)
    w2 = w2_ref[...]
    h2 = (aggA_ref[...] * dn) @ w2[0:128, :] + (aggB_ref[...] * dn) @ w2[128:256, :]
    h2 = jnp.maximum(h2 + b2_ref[0], 0.0) * _src_norm(d)
    t = jnp.dot(h2, w3_ref[...], preferred_element_type=f32)

    @pl.when(c == 0)
    def _():
        out_ref[...] = jnp.concatenate([t, jnp.zeros((1000, 64), f32)], axis=1)

    @pl.when(c == 1)
    def _():
        out_ref[...] = jnp.zeros((1000, 128), f32)


def _l2t_call(agg2, degs, W2, b2, W3):
    return pl.pallas_call(
        _l2t_body,
        grid=(2, 10),
        in_specs=[
            pl.BlockSpec((1000, 128), lambda c, r: (r, 0)),
            pl.BlockSpec((1000, 128), lambda c, r: (10 + r, 0)),
            pl.BlockSpec((1000, 64), lambda c, r: (r, 0)),
            pl.BlockSpec((256, 256), lambda c, r: (0, 0)),
            pl.BlockSpec((1, 256), lambda c, r: (0, 0)),
            pl.BlockSpec((256, 64), lambda c, r: (0, 0)),
        ],
        out_specs=pl.BlockSpec((1000, 128), lambda c, r: (c * 10 + r, 0)),
        out_shape=jax.ShapeDtypeStruct((2 * NN, 128), f32),
    )(agg2, agg2, degs, W2, b2.reshape(1, 256), W3)


def _final_body(pa_ref, deg_ref, b_ref, out_ref):
    d = deg_ref[...]
    x = (pa_ref[...] * _dst_norm(d))[:, :64] + b_ref[0]
    out_ref[...] = 1.0 / (1.0 + jnp.exp(-x)) + 1e-8


def _final_call(p, degs, b3):
    return pl.pallas_call(
        _final_body,
        grid=(10,),
        in_specs=[
            pl.BlockSpec((1000, 128), lambda r: (r, 0)),
            pl.BlockSpec((1000, 64), lambda r: (r, 0)),
            pl.BlockSpec((1, 64), lambda r: (0, 0)),
        ],
        out_specs=pl.BlockSpec((1000, 64), lambda r: (r, 0)),
        out_shape=jax.ShapeDtypeStruct((NN, 64), f32),
    )(p, degs, b3.reshape(1, 64))


def kernel(features, edge_index, W1, b1, W2, b2, W3, b3):
    src = edge_index[0]
    dst = edge_index[1]
    src_plus = src + NN

    degs = _deg_call(src, dst).reshape(64, NN).transpose(1, 0)
    h_s = _prep_call(features, degs)
    agg1 = _agg_call(h_s, src, src_plus, dst)
    h1s = _layer_call(agg1, degs, W1, b1)
    agg2 = _agg_call(h1s, src, src_plus, dst)
    t = _l2t_call(agg2, degs, W2, b2, W3)
    p = _agg_call(t, src, src_plus, dst)
    return _final_call(p, degs, b3)


# speedup vs baseline: 10.1408x; 1.3076x over previous
import jax
import jax.numpy as jnp
from jax import lax
from jax.experimental import pallas as pl
from jax.experimental.pallas import tpu as pltpu
from jax.experimental.pallas import tpu_sc as plsc

NN = 10000
EE = 160000
SB = 624
SL = 640
CB = 128

f32 = jnp.float32
i32 = jnp.int32


def _mesh():
    return plsc.VectorSubcoreMesh(core_axis_name="c", subcore_axis_name="s")


def _stripe(s, fn):
    @pl.when(s < 15)
    def _():
        fn(s * SB, SB)

    @pl.when(s == 15)
    def _():
        fn(15 * SB, SL)


def _deg_body(src_hbm, dst_hbm, out_hbm, out2_hbm, hist_s, hist_d, idx_v, idx_v2):
    c = lax.axis_index("c")
    s = lax.axis_index("s")
    w = c * 16 + s
    zz = jnp.zeros((16,), f32)

    def zloop(i, _):
        hist_s[pl.ds(i * 16, 16)] = zz
        hist_d[pl.ds(i * 16, 16)] = zz
        return 0

    lax.fori_loop(0, NN // 16, zloop, 0)
    ones_r = jnp.ones((16,), f32)

    def block(ebase, nedges):
        pltpu.sync_copy(src_hbm.at[pl.ds(ebase, nedges)],
                        idx_v.at[pl.ds(0, nedges)])
        pltpu.sync_copy(dst_hbm.at[pl.ds(ebase, nedges)],
                        idx_v2.at[pl.ds(0, nedges)])

        def group(i, _):
            sl = pl.ds(i * 16, 16)
            plsc.addupdate_scatter(hist_s, [idx_v[sl]], ones_r)
            plsc.addupdate_scatter(hist_d, [idx_v2[sl]], ones_r)
            return 0

        lax.fori_loop(0, nedges // 16, group, 0)

        def shift(i, _):
            sl = pl.ds(i * 16, 16)
            idx_v[sl] = idx_v[sl] + NN
            return 0

        lax.fori_loop(0, nedges // 16, shift, 0)
        pltpu.sync_copy(idx_v.at[pl.ds(0, nedges)],
                        out2_hbm.at[pl.ds(ebase, nedges)])

    def chunks(k, _):
        block(w * 4992 + k * 624, 624)
        return 0

    lax.fori_loop(0, 8, chunks, 0)

    @pl.when(w == 31)
    def _():
        block(31 * 4992 + 4992, 256)

    pltpu.sync_copy(hist_s, out_hbm.at[pl.ds(w * NN, NN)])
    pltpu.sync_copy(hist_d, out_hbm.at[pl.ds((32 + w) * NN, NN)])


def _deg_call(src, dst):
    return pl.kernel(
        _deg_body,
        out_type=(jax.ShapeDtypeStruct((64 * NN,), f32),
                  jax.ShapeDtypeStruct((EE,), i32)),
        mesh=_mesh(),
        compiler_params=pltpu.CompilerParams(needs_layout_passes=False),
        scratch_types=[
            pltpu.VMEM((NN,), f32),
            pltpu.VMEM((NN,), f32),
            pltpu.VMEM((624,), i32),
            pltpu.VMEM((624,), i32),
        ],
    )(src, dst)


CBA = 80
KSC = 25


def _agg_body(h_hbm, src0_hbm, src1_hbm, dst_hbm, out_hbm,
              acc, idxb, dstb, rowsb, isem, dsem, g0, g1):
    c = lax.axis_index("c")
    s = lax.axis_index("s")
    off = c * NN
    gsems = (g0, g1)

    def initf(base, n):
        pltpu.sync_copy(h_hbm.at[pl.ds(off + base, n)],
                        acc.at[pl.ds(base, n)])

    _stripe(s, initf)
    plsc.subcore_barrier()

    def superchunk(src_hbm, base):
        ic = pltpu.async_copy(src_hbm.at[pl.ds(base, KSC * CBA)],
                              idxb, isem)
        for j in range(KSC):
            pltpu.async_copy(dst_hbm.at[pl.ds(base + j * CBA, CBA)],
                             dstb.at[j], dsem)
        ic.wait()
        for j in range(2):
            pltpu.async_copy(
                h_hbm.at[idxb.at[pl.ds(j * CBA, CBA)]],
                rowsb.at[pl.ds(j * CBA, CBA)], gsems[j])
        for j in range(KSC):
            pltpu.make_async_copy(dst_hbm.at[pl.ds(base + j * CBA, CBA)],
                                  dstb.at[j], dsem).wait()

        def chpair(i, _):
            for b in range(2):
                j = 2 * i + b
                pltpu.make_async_copy(
                    h_hbm.at[idxb.at[pl.ds(j * CBA, CBA)]],
                    rowsb.at[pl.ds(b * CBA, CBA)], gsems[b]).wait()
                pltpu.sync_copy(rowsb.at[pl.ds(b * CBA, CBA)],
                                acc.at[dstb.at[j]], add=True)

                @pl.when(j + 2 < KSC)
                def _():
                    pltpu.async_copy(
                        h_hbm.at[idxb.at[pl.ds((j + 2) * CBA, CBA)]],
                        rowsb.at[pl.ds(b * CBA, CBA)], gsems[b])
            return 0

        lax.fori_loop(0, KSC // 2, chpair, 0)
        jt = KSC - 1
        pltpu.make_async_copy(
            h_hbm.at[idxb.at[pl.ds(0, CBA)]],
            rowsb.at[pl.ds(0, CBA)], gsems[0]).wait()
        pltpu.sync_copy(rowsb.at[pl.ds(0, CBA)],
                        acc.at[dstb.at[jt]], add=True)

    def run(src_hbm):
        ebase = s * 10000

        def sc_loop(g, _):
            superchunk(src_hbm, ebase + g * (KSC * CBA))
            return 0

        lax.fori_loop(0, 10000 // (KSC * CBA), sc_loop, 0)

    @pl.when(c == 0)
    def _():
        run(src0_hbm)

    @pl.when(c == 1)
    def _():
        run(src1_hbm)

    plsc.subcore_barrier()

    def drainf(base, n):
        pltpu.sync_copy(acc.at[pl.ds(base, n)],
                        out_hbm.at[pl.ds(off + base, n)])

    _stripe(s, drainf)


def _agg_call(h_flat, src, src_plus, dst):
    return pl.kernel(
        _agg_body,
        out_type=jax.ShapeDtypeStruct((2 * NN, 128), f32),
        mesh=_mesh(),
        scratch_types=[
            pltpu.VMEM_SHARED((NN, 128), f32),
            pltpu.VMEM((KSC * CBA,), i32),
            pltpu.VMEM((KSC, CBA), i32),
            pltpu.VMEM((2 * CBA, 128), f32),
            pltpu.SemaphoreType.DMA,
            pltpu.SemaphoreType.DMA,
            pltpu.SemaphoreType.DMA,
            pltpu.SemaphoreType.DMA,
        ],
    )(h_flat, src, src_plus, dst)


def _src_norm(n):
    return n[:, 0:1]


def _dst_norm(n):
    return n[:, 1:2]


def _prep_body(feat_ref, deg_ref, out_ref, norm_ref):
    d = deg_ref[...]
    sn = lax.rsqrt(jnp.sum(d[0:32], axis=0) + 1.0)
    dn = lax.rsqrt(jnp.sum(d[32:64], axis=0) + 1.0)
    norm_ref[...] = jnp.stack([sn, dn], axis=1)
    out_ref[...] = feat_ref[...] * sn[:, None]


def _prep_call(features, degs):
    return pl.pallas_call(
        _prep_body,
        grid=(2,),
        in_specs=[
            pl.BlockSpec((NN, 128), lambda c: (0, c)),
            pl.BlockSpec((64, NN), lambda c: (0, 0)),
        ],
        out_specs=[
            pl.BlockSpec((NN, 128), lambda c: (c, 0)),
            pl.BlockSpec((NN, 2), lambda c: (0, 0)),
        ],
        out_shape=[
            jax.ShapeDtypeStruct((2 * NN, 128), f32),
            jax.ShapeDtypeStruct((NN, 2), f32),
        ],
    )(features, degs)


def _layer_body(agg_ref, deg_ref, w_ref, b_ref, out_ref, acc_ref):
    k = pl.program_id(2)
    d = deg_ref[...]
    x = agg_ref[...] * _dst_norm(d)
    p = jnp.dot(x, w_ref[...], preferred_element_type=f32)

    @pl.when(k == 0)
    def _():
        acc_ref[...] = p

    @pl.when(k == 1)
    def _():
        a = acc_ref[...] + p + b_ref[0, 0]
        out_ref[...] = jnp.maximum(a, 0.0) * _src_norm(d)


def _layer_call(agg_flat, norms, W, b):
    return pl.pallas_call(
        _layer_body,
        grid=(2, 10, 2),
        in_specs=[
            pl.BlockSpec((1000, 128), lambda c, r, k: (k * 10 + r, 0)),
            pl.BlockSpec((1000, 2), lambda c, r, k: (r, 0)),
            pl.BlockSpec((128, 128), lambda c, r, k: (k, c)),
            pl.BlockSpec((1, 1, 128), lambda c, r, k: (c, 0, 0)),
        ],
        out_specs=pl.BlockSpec((1000, 128), lambda c, r, k: (c * 10 + r, 0)),
        out_shape=jax.ShapeDtypeStruct((2 * NN, 128), f32),
        scratch_shapes=[pltpu.VMEM((1000, 128), f32)],
    )(agg_flat, norms, W, b.reshape(2, 1, 128))


def _l2t_body(aggA_ref, aggB_ref, deg_ref, w2_ref, b2_ref, w3_ref, out_ref):
    c = pl.program_id(0)
    d = deg_ref[...]
    dn = _dst_norm(d)
    w2 = w2_ref[...]
    h2 = (aggA_ref[...] * dn) @ w2[0:128, :] + (aggB_ref[...] * dn) @ w2[128:256, :]
    h2 = jnp.maximum(h2 + b2_ref[0], 0.0) * _src_norm(d)
    t = jnp.dot(h2, w3_ref[...], preferred_element_type=f32)

    @pl.when(c == 0)
    def _():
        out_ref[...] = jnp.concatenate([t, jnp.zeros((1000, 64), f32)], axis=1)

    @pl.when(c == 1)
    def _():
        out_ref[...] = jnp.zeros((1000, 128), f32)


def _l2t_call(agg2, norms, W2, b2, W3):
    return pl.pallas_call(
        _l2t_body,
        grid=(2, 10),
        in_specs=[
            pl.BlockSpec((1000, 128), lambda c, r: (r, 0)),
            pl.BlockSpec((1000, 128), lambda c, r: (10 + r, 0)),
            pl.BlockSpec((1000, 2), lambda c, r: (r, 0)),
            pl.BlockSpec((256, 256), lambda c, r: (0, 0)),
            pl.BlockSpec((1, 256), lambda c, r: (0, 0)),
            pl.BlockSpec((256, 64), lambda c, r: (0, 0)),
        ],
        out_specs=pl.BlockSpec((1000, 128), lambda c, r: (c * 10 + r, 0)),
        out_shape=jax.ShapeDtypeStruct((2 * NN, 128), f32),
    )(agg2, agg2, norms, W2, b2.reshape(1, 256), W3)


def _final_body(pa_ref, deg_ref, b_ref, out_ref):
    d = deg_ref[...]
    x = (pa_ref[...] * _dst_norm(d))[:, :64] + b_ref[0]
    out_ref[...] = 1.0 / (1.0 + jnp.exp(-x)) + 1e-8


def _final_call(p, norms, b3):
    return pl.pallas_call(
        _final_body,
        grid=(10,),
        in_specs=[
            pl.BlockSpec((1000, 128), lambda r: (r, 0)),
            pl.BlockSpec((1000, 2), lambda r: (r, 0)),
            pl.BlockSpec((1, 64), lambda r: (0, 0)),
        ],
        out_specs=pl.BlockSpec((1000, 64), lambda r: (r, 0)),
        out_shape=jax.ShapeDtypeStruct((NN, 64), f32),
    )(p, norms, b3.reshape(1, 64))


def kernel(features, edge_index, W1, b1, W2, b2, W3, b3):
    src = edge_index[0]
    dst = edge_index[1]

    degp, src_plus = _deg_call(src, dst)
    degs = degp.reshape(64, NN)
    h_s, norms = _prep_call(features, degs)
    agg1 = _agg_call(h_s, src, src_plus, dst)
    h1s = _layer_call(agg1, norms, W1, b1)
    agg2 = _agg_call(h1s, src, src_plus, dst)
    t = _l2t_call(agg2, norms, W2, b2, W3)
    p = _agg_call(t, src, src_plus, dst)
    return _final_call(p, norms, b3)
